# baseline (speedup 1.0000x reference)
import functools

import jax
import jax.numpy as jnp
from jax import lax
from jax.experimental import pallas as pl
from jax.experimental.pallas import tpu as pltpu
from jax.experimental.pallas import tpu_sc as plsc

N = 10000
E = 160000
S = 256
R = 9
ED = 20
GEB = 2

BN = 400
BE = 1600

MC = 4 * S
NPAD = 10240
PER_SC = NPAD // 2
NCH = 10
CH = 512
ACC_ROWS = CH + 16
DUMMY = CH
NW = 32


def _phi_body(s_ref, w1_ref, b1_ref, w2_ref, b2_ref, o_ref):
    x = s_ref[...]
    h1 = jax.nn.silu(x @ w1_ref[...] + b1_ref[...][None, :])
    o_ref[...] = h1 @ w2_ref[...] + b2_ref[...][None, :]


def _mlp2_pallas(x, W1, b1, W2, b2, block_rows):
    n, _ = x.shape
    d_out = W2.shape[1]
    grid = (n // block_rows,)
    return pl.pallas_call(
        _phi_body,
        grid=grid,
        in_specs=[
            pl.BlockSpec((block_rows, x.shape[1]), lambda i: (i, 0)),
            pl.BlockSpec(W1.shape, lambda i: (0, 0)),
            pl.BlockSpec(b1.shape, lambda i: (0,)),
            pl.BlockSpec(W2.shape, lambda i: (0, 0)),
            pl.BlockSpec(b2.shape, lambda i: (0,)),
        ],
        out_specs=pl.BlockSpec((block_rows, d_out), lambda i: (i, 0)),
        out_shape=jax.ShapeDtypeStruct((n, d_out), jnp.float32),
    )(x, W1, b1, W2, b2)


def _we_body(rbf_ref, wr_ref, br_ref, o_ref):
    o_ref[...] = rbf_ref[...] @ wr_ref[...] + br_ref[...][None, :]


def _we_pallas(rbf, Wr, br):
    grid = (E // BE,)
    return pl.pallas_call(
        _we_body,
        grid=grid,
        in_specs=[
            pl.BlockSpec((BE, ED), lambda i: (i, 0)),
            pl.BlockSpec(Wr.shape, lambda i: (0, 0)),
            pl.BlockSpec(br.shape, lambda i: (0,)),
        ],
        out_specs=pl.BlockSpec((BE, 3 * S), lambda i: (i, 0)),
        out_shape=jax.ShapeDtypeStruct((E, 3 * S), jnp.float32),
    )(rbf, Wr, br)


def _embed_body(h_ref, w_ref, b_ref, ctx_ref, o_ref):
    o_ref[...] = h_ref[...] @ w_ref[...] + b_ref[...][None, :] + ctx_ref[...]


def _embed_pallas(h, W, b, ctx):
    grid = (N // BN,)
    return pl.pallas_call(
        _embed_body,
        grid=grid,
        in_specs=[
            pl.BlockSpec((BN, 5), lambda i: (i, 0)),
            pl.BlockSpec(W.shape, lambda i: (0, 0)),
            pl.BlockSpec(b.shape, lambda i: (0,)),
            pl.BlockSpec((1, S), lambda i: (0, 0)),
        ],
        out_specs=pl.BlockSpec((BN, S), lambda i: (i, 0)),
        out_shape=jax.ShapeDtypeStruct((N, S), jnp.float32),
    )(h, W, b, ctx)


def _update_body(s_ref, v_ref, agg_ref, ctx_ref,
                 u_ref, vw_ref, w1_ref, b1_ref, w2_ref, b2_ref,
                 so_ref, vo_ref):
    agg = agg_ref[...]
    s1 = s_ref[...] + agg[:, 0:S]
    v1 = v_ref[...] + agg[:, S:4 * S]
    U = u_ref[...]
    Vw = vw_ref[...]
    uv0 = v1[:, 0 * S:1 * S] @ U
    uv1 = v1[:, 1 * S:2 * S] @ U
    uv2 = v1[:, 2 * S:3 * S] @ U
    vv0 = v1[:, 0 * S:1 * S] @ Vw
    vv1 = v1[:, 1 * S:2 * S] @ Vw
    vv2 = v1[:, 2 * S:3 * S] @ Vw
    vn = jnp.sqrt(vv0 * vv0 + vv1 * vv1 + vv2 * vv2 + 1e-8)
    pre = s1 @ w1_ref[0:S, :] + vn @ w1_ref[S:2 * S, :] + b1_ref[...][None, :]
    a = jax.nn.silu(pre) @ w2_ref[...] + b2_ref[...][None, :]
    dot = uv0 * vv0 + uv1 * vv1 + uv2 * vv2
    a_vv = a[:, 2 * S:3 * S]
    so_ref[...] = (s1 + a[:, 0:S] + a[:, S:2 * S] * dot + ctx_ref[...])
    vo_ref[...] = v1 + jnp.concatenate(
        [a_vv * uv0, a_vv * uv1, a_vv * uv2], axis=1)


def _update_pallas(s, v_cat, agg, ctx, U, Vw, W1, b1, W2, b2):
    grid = (N // BN,)
    return pl.pallas_call(
        _update_body,
        grid=grid,
        in_specs=[
            pl.BlockSpec((BN, S), lambda i: (i, 0)),
            pl.BlockSpec((BN, 3 * S), lambda i: (i, 0)),
            pl.BlockSpec((BN, MC), lambda i: (i, 0)),
            pl.BlockSpec((1, S), lambda i: (0, 0)),
            pl.BlockSpec(U.shape, lambda i: (0, 0)),
            pl.BlockSpec(Vw.shape, lambda i: (0, 0)),
            pl.BlockSpec(W1.shape, lambda i: (0, 0)),
            pl.BlockSpec(b1.shape, lambda i: (0,)),
            pl.BlockSpec(W2.shape, lambda i: (0, 0)),
            pl.BlockSpec(b2.shape, lambda i: (0,)),
        ],
        out_specs=[
            pl.BlockSpec((BN, S), lambda i: (i, 0)),
            pl.BlockSpec((BN, 3 * S), lambda i: (i, 0)),
        ],
        out_shape=[
            jax.ShapeDtypeStruct((N, S), jnp.float32),
            jax.ShapeDtypeStruct((N, 3 * S), jnp.float32),
        ],
    )(s, v_cat, agg, ctx, U, Vw, W1, b1, W2, b2)


def _geb_body(s_ref, v_ref, ctx_ref, wv1_ref, wv2_ref,
              w1_ref, b1_ref, w2_ref, b2_ref, so_ref, vo_ref):
    v = v_ref[...]
    Wv1 = wv1_ref[...]
    Wv2 = wv2_ref[...]
    v10 = v[:, 0 * S:1 * S] @ Wv1
    v11 = v[:, 1 * S:2 * S] @ Wv1
    v12 = v[:, 2 * S:3 * S] @ Wv1
    v20 = v[:, 0 * S:1 * S] @ Wv2
    v21 = v[:, 1 * S:2 * S] @ Wv2
    v22 = v[:, 2 * S:3 * S] @ Wv2
    n2 = jnp.sqrt(v20 * v20 + v21 * v21 + v22 * v22 + 1e-8)
    pre = s_ref[...] @ w1_ref[0:S, :] + n2 @ w1_ref[S:2 * S, :] + b1_ref[...][None, :]
    xg = jax.nn.silu(pre) @ w2_ref[...] + b2_ref[...][None, :]
    gate = xg[:, S:2 * S]
    so_ref[...] = xg[:, 0:S] + ctx_ref[...]
    vo_ref[...] = jnp.concatenate([gate * v10, gate * v11, gate * v12], axis=1)


def _geb_pallas(s, v_cat, ctx, Wv1, Wv2, W1, b1, W2, b2):
    grid = (N // BN,)
    return pl.pallas_call(
        _geb_body,
        grid=grid,
        in_specs=[
            pl.BlockSpec((BN, S), lambda i: (i, 0)),
            pl.BlockSpec((BN, 3 * S), lambda i: (i, 0)),
            pl.BlockSpec((1, S), lambda i: (0, 0)),
            pl.BlockSpec(Wv1.shape, lambda i: (0, 0)),
            pl.BlockSpec(Wv2.shape, lambda i: (0, 0)),
            pl.BlockSpec(W1.shape, lambda i: (0, 0)),
            pl.BlockSpec(b1.shape, lambda i: (0,)),
            pl.BlockSpec(W2.shape, lambda i: (0, 0)),
            pl.BlockSpec(b2.shape, lambda i: (0,)),
        ],
        out_specs=[
            pl.BlockSpec((BN, S), lambda i: (i, 0)),
            pl.BlockSpec((BN, 3 * S), lambda i: (i, 0)),
        ],
        out_shape=[
            jax.ShapeDtypeStruct((N, S), jnp.float32),
            jax.ShapeDtypeStruct((N, 3 * S), jnp.float32),
        ],
    )(s, v_cat, ctx, Wv1, Wv2, W1, b1, W2, b2)


def _readout_body(s_ref, v_ref, iw1_ref, ib1_ref, iw2_ref, ib2_ref,
                  ew1_ref, eb1_ref, ew2_ref, eb2_ref, wvec_ref,
                  eo_ref, io_ref):
    s = s_ref[...]
    v = v_ref[...]
    inv = jax.nn.silu(s @ iw1_ref[...] + ib1_ref[...][None, :]) @ iw2_ref[...] \
        + ib2_ref[...][None, :]
    gate = jax.nn.silu(s @ ew1_ref[...] + eb1_ref[...][None, :]) @ ew2_ref[...] \
        + eb2_ref[...][None, :]
    wv = wvec_ref[...]
    vec0 = jnp.sum(v[:, 0 * S:1 * S] * wv, axis=1, keepdims=True)
    vec1 = jnp.sum(v[:, 1 * S:2 * S] * wv, axis=1, keepdims=True)
    vec2 = jnp.sum(v[:, 2 * S:3 * S] * wv, axis=1, keepdims=True)
    eo_ref[...] = gate * jnp.concatenate([vec0, vec1, vec2], axis=1)
    io_ref[...] = inv


def _readout_pallas(s, v_cat, p):
    grid = (N // BN,)
    wvec = p['equi_wvec'][None, :]
    return pl.pallas_call(
        _readout_body,
        grid=grid,
        in_specs=[
            pl.BlockSpec((BN, S), lambda i: (i, 0)),
            pl.BlockSpec((BN, 3 * S), lambda i: (i, 0)),
            pl.BlockSpec(p['inv_W1'].shape, lambda i: (0, 0)),
            pl.BlockSpec(p['inv_b1'].shape, lambda i: (0,)),
            pl.BlockSpec(p['inv_W2'].shape, lambda i: (0, 0)),
            pl.BlockSpec(p['inv_b2'].shape, lambda i: (0,)),
            pl.BlockSpec(p['equi_W1'].shape, lambda i: (0, 0)),
            pl.BlockSpec(p['equi_b1'].shape, lambda i: (0,)),
            pl.BlockSpec(p['equi_W2'].shape, lambda i: (0, 0)),
            pl.BlockSpec(p['equi_b2'].shape, lambda i: (0,)),
            pl.BlockSpec((1, S), lambda i: (0, 0)),
        ],
        out_specs=[
            pl.BlockSpec((BN, 3), lambda i: (i, 0)),
            pl.BlockSpec((BN, 5), lambda i: (i, 0)),
        ],
        out_shape=[
            jax.ShapeDtypeStruct((N, 3), jnp.float32),
            jax.ShapeDtypeStruct((N, 5), jnp.float32),
        ],
    )(s, v_cat, p['inv_W1'], p['inv_b1'], p['inv_W2'], p['inv_b2'],
      p['equi_W1'], p['equi_b1'], p['equi_W2'], p['equi_b2'], wvec)


def _rbf_body(rij_ref, dirb_ref, rbf_ref):
    rij = rij_ref[...]
    d2 = jnp.sum(rij * rij, axis=1, keepdims=True)
    d = jnp.maximum(jnp.sqrt(d2), 1e-6)
    dirv = rij / d
    dirb_ref[...] = jnp.broadcast_to(dirv[:, :, None], (BE, 3, 16)).reshape(BE, 48)
    ci = lax.broadcasted_iota(jnp.int32, (1, ED), 1)
    centers = ci.astype(jnp.float32) * (5.0 / (ED - 1))
    rbf_ref[...] = jnp.exp(-10.0 * (d - centers) ** 2)


def _rbf_pallas(r_ij):
    grid = (E // BE,)
    return pl.pallas_call(
        _rbf_body,
        grid=grid,
        in_specs=[pl.BlockSpec((BE, 3), lambda i: (i, 0))],
        out_specs=[
            pl.BlockSpec((BE, 48), lambda i: (i, 0)),
            pl.BlockSpec((BE, ED), lambda i: (i, 0)),
        ],
        out_shape=[
            jax.ShapeDtypeStruct((E, 48), jnp.float32),
            jax.ShapeDtypeStruct((E, ED), jnp.float32),
        ],
    )(r_ij)


def _edge_sc_body(phi, v, we, dirb, rowi, coli, wb, agg,
                  idxb, colraw, colbuf, phib, vb, web, dirbb, msgb,
                  zbuf, wbp, acc, sem):
    sc = lax.axis_index("c")
    tile = lax.axis_index("s")
    w = sc * 16 + tile
    iota = lax.iota(jnp.int32, 16)
    zero16 = jnp.zeros((16,), jnp.float32)
    rows = CH // 16

    def zzr(i, _):
        zbuf[i // 8, pl.ds((i % 8) * 16, 16)] = zero16
        return 0
    lax.fori_loop(0, 128 * 8, zzr, 0)

    def one_pass(p, _):
        pltpu.sync_copy(wb.at[w, p], wbp)
        wbv = wbp[...]
        a0 = wbv[0]
        nb = wbv[1]
        a = wbv[2]
        b = wbv[3]
        base = sc * PER_SC + p * CH

        for jz in range(rows * 8 // 128):
            pltpu.sync_copy(zbuf, acc.at[pl.ds(tile * rows * 8 + jz * 128, 128)])
        plsc.subcore_barrier()

        def blk(i, __):
            e0 = pl.multiple_of(a0 + i * 16, 16)
            pltpu.sync_copy(rowi.at[pl.ds(e0, 16)], idxb)
            pltpu.sync_copy(coli.at[pl.ds(e0, 16)], colraw)
            c1 = pltpu.async_copy(phi.at[idxb], phib, sem)
            c2 = pltpu.async_copy(v.at[idxb], vb, sem)
            c3 = pltpu.async_copy(we.at[pl.ds(e0, 16)], web, sem)
            c4 = pltpu.async_copy(dirb.at[pl.ds(e0, 16)], dirbb, sem)
            colv = colraw[...]
            evec = e0 + iota
            lcol = colv - base
            ok = (evec >= a) & (evec < b) & (lcol >= 0) & (lcol < CH)
            lcolm = jnp.where(ok, lcol, DUMMY)
            for c in range(8):
                lo = lcolm[2 * c] * 8
                hi = lcolm[2 * c + 1] * 8
                colbuf[pl.ds(c * 16, 16)] = jnp.where(
                    iota < 8, lo + iota, hi + (iota - 8))
            c1.wait()
            c2.wait()
            c3.wait()
            c4.wait()

            def edge4(jj, ___):
                for dj in range(8):
                    j = jj * 8 + dj
                    r8 = j * 8
                    d0 = dirbb[j, pl.ds(0, 16)]
                    d1 = dirbb[j, pl.ds(16, 16)]
                    d2 = dirbb[j, pl.ds(32, 16)]
                    for k in range(16):
                        o = k * 16

                        def mst(off, val):
                            msgb[r8 + off // 128, pl.ds(off % 128, 16)] = val
                        mst(o, phib[j, pl.ds(o, 16)] * web[j, pl.ds(o, 16)])
                        t1 = phib[j, pl.ds(256 + o, 16)] * web[j, pl.ds(256 + o, 16)]
                        t2 = phib[j, pl.ds(512 + o, 16)] * web[j, pl.ds(512 + o, 16)]
                        vr0 = vb[j, pl.ds(o, 16)]
                        vr1 = vb[j, pl.ds(256 + o, 16)]
                        vr2 = vb[j, pl.ds(512 + o, 16)]
                        mst(256 + o, t1 * vr0 + t2 * d0)
                        mst(512 + o, t1 * vr1 + t2 * d1)
                        mst(768 + o, t1 * vr2 + t2 * d2)
                return 0
            lax.fori_loop(0, 2, edge4, 0)
            pltpu.sync_copy(msgb, acc.at[colbuf], add=True)
            return 0
        lax.fori_loop(0, nb, blk, 0)
        plsc.subcore_barrier()

        glo8 = pl.multiple_of((base + tile * rows) * 8, 8)
        pltpu.sync_copy(acc.at[pl.ds(tile * rows * 8, rows * 8)],
                        agg.at[pl.ds(glo8, rows * 8)])
        plsc.subcore_barrier()
        return 0
    lax.fori_loop(0, NCH, one_pass, 0)
    return None


_edge_sc = functools.partial(
    pl.kernel,
    mesh=plsc.VectorSubcoreMesh(core_axis_name="c", subcore_axis_name="s"),
    out_type=jax.ShapeDtypeStruct((NPAD * 8, 128), jnp.float32),
    scratch_types=[
        pltpu.VMEM((16,), jnp.int32),
        pltpu.VMEM((16,), jnp.int32),
        pltpu.VMEM((128,), jnp.int32),
        pltpu.VMEM((16, 3 * S), jnp.float32),
        pltpu.VMEM((16, 3 * S), jnp.float32),
        pltpu.VMEM((16, 3 * S), jnp.float32),
        pltpu.VMEM((16, 48), jnp.float32),
        pltpu.VMEM((128, 128), jnp.float32),
        pltpu.VMEM((128, 128), jnp.float32),
        pltpu.VMEM((16,), jnp.int32),
        pltpu.VMEM_SHARED((ACC_ROWS * 8, 128), jnp.float32),
        pltpu.SemaphoreType.DMA,
    ],
)(_edge_sc_body)


def _edge_prep(pos, row, col):
    perm = jnp.argsort(col)
    row_s = row[perm].astype(jnp.int32)
    col_s = col[perm].astype(jnp.int32)
    r_ij = pos[col_s] - pos[row_s]
    dirb, rbf_s = _rbf_pallas(r_ij)

    nodes = jnp.asarray(
        [sc * PER_SC + p * CH for sc in range(2) for p in range(NCH)],
        dtype=jnp.int32)
    cuts = jnp.searchsorted(col_s, nodes, side='left').astype(jnp.int32)
    clo = jnp.concatenate([cuts, jnp.asarray([E], dtype=jnp.int32)])

    wi = jnp.arange(NW, dtype=jnp.int32)
    sc = wi // 16
    t = wi % 16

    rows_wb = []
    for p in range(NCH):
        ca = clo[sc * NCH + p]
        cb = clo[sc * NCH + p + 1]
        a = ca + ((cb - ca) * t) // 16
        b = ca + ((cb - ca) * (t + 1)) // 16
        a0 = (a // 16) * 16
        nb = jnp.where(b > a, (b - a0 + 15) // 16, 0)
        z = jnp.zeros_like(wi)
        rows_wb.append(jnp.stack([a0, nb, a, b] + [z] * 12, axis=1))
    WB = jnp.stack(rows_wb, axis=1).astype(jnp.int32)
    return row_s, col_s, rbf_s, dirb, WB


def kernel(h, pos, edge_index, t, params):
    p = params
    row = edge_index[0]
    col = edge_index[1]

    row_s, col_s, rbf_s, dirb, WB = _edge_prep(pos, row, col)

    half = S // 2
    freqs = jnp.exp(-jnp.log(10000.0) * jnp.arange(half) / half)
    ang = t[:, None] * freqs[None, :]
    ctx = jax.nn.silu(
        jnp.concatenate([jnp.sin(ang), jnp.cos(ang)], axis=1)
        @ p['time_W1'] + p['time_b1']) @ p['time_W2'] + p['time_b2']

    s = _embed_pallas(h, p['emb_W'], p['emb_b'], ctx)
    v_cat = jnp.zeros((N, 3 * S), dtype=jnp.float32)

    for r in range(R):
        phi = _mlp2_pallas(s, p['msg_W1'][r], p['msg_b1'][r],
                           p['msg_W2'][r], p['msg_b2'][r], BN)
        we_s = _we_pallas(rbf_s, p['msg_Wr'][r], p['msg_br'][r])
        agg = _edge_sc(phi, v_cat, we_s, dirb, row_s, col_s, WB)
        agg = agg.reshape(NPAD, MC)
        s, v_cat = _update_pallas(s, v_cat, agg, ctx,
                                  p['upd_U'][r], p['upd_V'][r],
                                  p['upd_W1'][r], p['upd_b1'][r],
                                  p['upd_W2'][r], p['upd_b2'][r])

    for g in range(GEB):
        s, v_cat = _geb_pallas(s, v_cat, ctx,
                               p['geb_Wv1'][g], p['geb_Wv2'][g],
                               p['geb_W1'][g], p['geb_b1'][g],
                               p['geb_W2'][g], p['geb_b2'][g])

    equi_out, inv_out = _readout_pallas(s, v_cat, p)
    return (equi_out, inv_out)


# speedup vs baseline: 1.4989x; 1.4989x over previous
import functools

import jax
import jax.numpy as jnp
from jax import lax
from jax.experimental import pallas as pl
from jax.experimental.pallas import tpu as pltpu
from jax.experimental.pallas import tpu_sc as plsc

N = 10000
E = 160000
S = 256
R = 9
ED = 20
GEB = 2

BN = 400
BE = 1600

MC = 4 * S
NPAD = 10240
PER_SC = NPAD // 2
NCH = 10
CH = 512
ACC_ROWS = CH + 16
DUMMY = CH
NW = 32


def _phi_body(s_ref, w1_ref, b1_ref, w2_ref, b2_ref, o_ref):
    x = s_ref[...]
    h1 = jax.nn.silu(x @ w1_ref[...] + b1_ref[...][None, :])
    o_ref[...] = h1 @ w2_ref[...] + b2_ref[...][None, :]


def _mlp2_pallas(x, W1, b1, W2, b2, block_rows):
    n, _ = x.shape
    d_out = W2.shape[1]
    grid = (n // block_rows,)
    return pl.pallas_call(
        _phi_body,
        grid=grid,
        in_specs=[
            pl.BlockSpec((block_rows, x.shape[1]), lambda i: (i, 0)),
            pl.BlockSpec(W1.shape, lambda i: (0, 0)),
            pl.BlockSpec(b1.shape, lambda i: (0,)),
            pl.BlockSpec(W2.shape, lambda i: (0, 0)),
            pl.BlockSpec(b2.shape, lambda i: (0,)),
        ],
        out_specs=pl.BlockSpec((block_rows, d_out), lambda i: (i, 0)),
        out_shape=jax.ShapeDtypeStruct((n, d_out), jnp.float32),
    )(x, W1, b1, W2, b2)


def _we_body(rbf_ref, wr_ref, br_ref, o_ref):
    o_ref[...] = rbf_ref[...] @ wr_ref[...] + br_ref[...][None, :]


def _we_pallas(rbf, Wr, br):
    grid = (E // BE,)
    return pl.pallas_call(
        _we_body,
        grid=grid,
        in_specs=[
            pl.BlockSpec((BE, ED), lambda i: (i, 0)),
            pl.BlockSpec(Wr.shape, lambda i: (0, 0)),
            pl.BlockSpec(br.shape, lambda i: (0,)),
        ],
        out_specs=pl.BlockSpec((BE, 3 * S), lambda i: (i, 0)),
        out_shape=jax.ShapeDtypeStruct((E, 3 * S), jnp.float32),
    )(rbf, Wr, br)


def _embed_body(h_ref, w_ref, b_ref, ctx_ref, o_ref):
    o_ref[...] = h_ref[...] @ w_ref[...] + b_ref[...][None, :] + ctx_ref[...]


def _embed_pallas(h, W, b, ctx):
    grid = (N // BN,)
    return pl.pallas_call(
        _embed_body,
        grid=grid,
        in_specs=[
            pl.BlockSpec((BN, 5), lambda i: (i, 0)),
            pl.BlockSpec(W.shape, lambda i: (0, 0)),
            pl.BlockSpec(b.shape, lambda i: (0,)),
            pl.BlockSpec((1, S), lambda i: (0, 0)),
        ],
        out_specs=pl.BlockSpec((BN, S), lambda i: (i, 0)),
        out_shape=jax.ShapeDtypeStruct((N, S), jnp.float32),
    )(h, W, b, ctx)


def _update_body(s_ref, v_ref, agg_ref, ctx_ref,
                 u_ref, vw_ref, w1_ref, b1_ref, w2_ref, b2_ref,
                 so_ref, vo_ref):
    agg = agg_ref[...]
    s1 = s_ref[...] + agg[:, 0:S]
    v1 = v_ref[...] + agg[:, S:4 * S]
    U = u_ref[...]
    Vw = vw_ref[...]
    uv0 = v1[:, 0 * S:1 * S] @ U
    uv1 = v1[:, 1 * S:2 * S] @ U
    uv2 = v1[:, 2 * S:3 * S] @ U
    vv0 = v1[:, 0 * S:1 * S] @ Vw
    vv1 = v1[:, 1 * S:2 * S] @ Vw
    vv2 = v1[:, 2 * S:3 * S] @ Vw
    vn = jnp.sqrt(vv0 * vv0 + vv1 * vv1 + vv2 * vv2 + 1e-8)
    pre = s1 @ w1_ref[0:S, :] + vn @ w1_ref[S:2 * S, :] + b1_ref[...][None, :]
    a = jax.nn.silu(pre) @ w2_ref[...] + b2_ref[...][None, :]
    dot = uv0 * vv0 + uv1 * vv1 + uv2 * vv2
    a_vv = a[:, 2 * S:3 * S]
    so_ref[...] = (s1 + a[:, 0:S] + a[:, S:2 * S] * dot + ctx_ref[...])
    vo_ref[...] = v1 + jnp.concatenate(
        [a_vv * uv0, a_vv * uv1, a_vv * uv2], axis=1)


def _update_pallas(s, v_cat, agg, ctx, U, Vw, W1, b1, W2, b2):
    grid = (N // BN,)
    return pl.pallas_call(
        _update_body,
        grid=grid,
        in_specs=[
            pl.BlockSpec((BN, S), lambda i: (i, 0)),
            pl.BlockSpec((BN, 3 * S), lambda i: (i, 0)),
            pl.BlockSpec((BN, MC), lambda i: (i, 0)),
            pl.BlockSpec((1, S), lambda i: (0, 0)),
            pl.BlockSpec(U.shape, lambda i: (0, 0)),
            pl.BlockSpec(Vw.shape, lambda i: (0, 0)),
            pl.BlockSpec(W1.shape, lambda i: (0, 0)),
            pl.BlockSpec(b1.shape, lambda i: (0,)),
            pl.BlockSpec(W2.shape, lambda i: (0, 0)),
            pl.BlockSpec(b2.shape, lambda i: (0,)),
        ],
        out_specs=[
            pl.BlockSpec((BN, S), lambda i: (i, 0)),
            pl.BlockSpec((BN, 3 * S), lambda i: (i, 0)),
        ],
        out_shape=[
            jax.ShapeDtypeStruct((N, S), jnp.float32),
            jax.ShapeDtypeStruct((N, 3 * S), jnp.float32),
        ],
    )(s, v_cat, agg, ctx, U, Vw, W1, b1, W2, b2)


def _geb_body(s_ref, v_ref, ctx_ref, wv1_ref, wv2_ref,
              w1_ref, b1_ref, w2_ref, b2_ref, so_ref, vo_ref):
    v = v_ref[...]
    Wv1 = wv1_ref[...]
    Wv2 = wv2_ref[...]
    v10 = v[:, 0 * S:1 * S] @ Wv1
    v11 = v[:, 1 * S:2 * S] @ Wv1
    v12 = v[:, 2 * S:3 * S] @ Wv1
    v20 = v[:, 0 * S:1 * S] @ Wv2
    v21 = v[:, 1 * S:2 * S] @ Wv2
    v22 = v[:, 2 * S:3 * S] @ Wv2
    n2 = jnp.sqrt(v20 * v20 + v21 * v21 + v22 * v22 + 1e-8)
    pre = s_ref[...] @ w1_ref[0:S, :] + n2 @ w1_ref[S:2 * S, :] + b1_ref[...][None, :]
    xg = jax.nn.silu(pre) @ w2_ref[...] + b2_ref[...][None, :]
    gate = xg[:, S:2 * S]
    so_ref[...] = xg[:, 0:S] + ctx_ref[...]
    vo_ref[...] = jnp.concatenate([gate * v10, gate * v11, gate * v12], axis=1)


def _geb_pallas(s, v_cat, ctx, Wv1, Wv2, W1, b1, W2, b2):
    grid = (N // BN,)
    return pl.pallas_call(
        _geb_body,
        grid=grid,
        in_specs=[
            pl.BlockSpec((BN, S), lambda i: (i, 0)),
            pl.BlockSpec((BN, 3 * S), lambda i: (i, 0)),
            pl.BlockSpec((1, S), lambda i: (0, 0)),
            pl.BlockSpec(Wv1.shape, lambda i: (0, 0)),
            pl.BlockSpec(Wv2.shape, lambda i: (0, 0)),
            pl.BlockSpec(W1.shape, lambda i: (0, 0)),
            pl.BlockSpec(b1.shape, lambda i: (0,)),
            pl.BlockSpec(W2.shape, lambda i: (0, 0)),
            pl.BlockSpec(b2.shape, lambda i: (0,)),
        ],
        out_specs=[
            pl.BlockSpec((BN, S), lambda i: (i, 0)),
            pl.BlockSpec((BN, 3 * S), lambda i: (i, 0)),
        ],
        out_shape=[
            jax.ShapeDtypeStruct((N, S), jnp.float32),
            jax.ShapeDtypeStruct((N, 3 * S), jnp.float32),
        ],
    )(s, v_cat, ctx, Wv1, Wv2, W1, b1, W2, b2)


def _readout_body(s_ref, v_ref, iw1_ref, ib1_ref, iw2_ref, ib2_ref,
                  ew1_ref, eb1_ref, ew2_ref, eb2_ref, wvec_ref,
                  eo_ref, io_ref):
    s = s_ref[...]
    v = v_ref[...]
    inv = jax.nn.silu(s @ iw1_ref[...] + ib1_ref[...][None, :]) @ iw2_ref[...] \
        + ib2_ref[...][None, :]
    gate = jax.nn.silu(s @ ew1_ref[...] + eb1_ref[...][None, :]) @ ew2_ref[...] \
        + eb2_ref[...][None, :]
    wv = wvec_ref[...]
    vec0 = jnp.sum(v[:, 0 * S:1 * S] * wv, axis=1, keepdims=True)
    vec1 = jnp.sum(v[:, 1 * S:2 * S] * wv, axis=1, keepdims=True)
    vec2 = jnp.sum(v[:, 2 * S:3 * S] * wv, axis=1, keepdims=True)
    eo_ref[...] = gate * jnp.concatenate([vec0, vec1, vec2], axis=1)
    io_ref[...] = inv


def _readout_pallas(s, v_cat, p):
    grid = (N // BN,)
    wvec = p['equi_wvec'][None, :]
    return pl.pallas_call(
        _readout_body,
        grid=grid,
        in_specs=[
            pl.BlockSpec((BN, S), lambda i: (i, 0)),
            pl.BlockSpec((BN, 3 * S), lambda i: (i, 0)),
            pl.BlockSpec(p['inv_W1'].shape, lambda i: (0, 0)),
            pl.BlockSpec(p['inv_b1'].shape, lambda i: (0,)),
            pl.BlockSpec(p['inv_W2'].shape, lambda i: (0, 0)),
            pl.BlockSpec(p['inv_b2'].shape, lambda i: (0,)),
            pl.BlockSpec(p['equi_W1'].shape, lambda i: (0, 0)),
            pl.BlockSpec(p['equi_b1'].shape, lambda i: (0,)),
            pl.BlockSpec(p['equi_W2'].shape, lambda i: (0, 0)),
            pl.BlockSpec(p['equi_b2'].shape, lambda i: (0,)),
            pl.BlockSpec((1, S), lambda i: (0, 0)),
        ],
        out_specs=[
            pl.BlockSpec((BN, 3), lambda i: (i, 0)),
            pl.BlockSpec((BN, 5), lambda i: (i, 0)),
        ],
        out_shape=[
            jax.ShapeDtypeStruct((N, 3), jnp.float32),
            jax.ShapeDtypeStruct((N, 5), jnp.float32),
        ],
    )(s, v_cat, p['inv_W1'], p['inv_b1'], p['inv_W2'], p['inv_b2'],
      p['equi_W1'], p['equi_b1'], p['equi_W2'], p['equi_b2'], wvec)


def _rbf_body(rij_ref, dirb_ref, rbf_ref):
    rij = rij_ref[...]
    d2 = jnp.sum(rij * rij, axis=1, keepdims=True)
    d = jnp.maximum(jnp.sqrt(d2), 1e-6)
    dirv = rij / d
    dirb_ref[...] = jnp.broadcast_to(dirv[:, :, None], (BE, 3, 16)).reshape(BE, 48)
    ci = lax.broadcasted_iota(jnp.int32, (1, ED), 1)
    centers = ci.astype(jnp.float32) * (5.0 / (ED - 1))
    rbf_ref[...] = jnp.exp(-10.0 * (d - centers) ** 2)


def _rbf_pallas(r_ij):
    grid = (E // BE,)
    return pl.pallas_call(
        _rbf_body,
        grid=grid,
        in_specs=[pl.BlockSpec((BE, 3), lambda i: (i, 0))],
        out_specs=[
            pl.BlockSpec((BE, 48), lambda i: (i, 0)),
            pl.BlockSpec((BE, ED), lambda i: (i, 0)),
        ],
        out_shape=[
            jax.ShapeDtypeStruct((E, 48), jnp.float32),
            jax.ShapeDtypeStruct((E, ED), jnp.float32),
        ],
    )(r_ij)


def _edge_sc_body(phi, v, we, dirb, rowi, coli, wb, agg,
                  idxb, colraw, colbuf, phib, vb, web, dirbb, msgb,
                  zbuf, wbp, acc, sem):
    sc = lax.axis_index("c")
    tile = lax.axis_index("s")
    w = sc * 16 + tile
    iota = lax.iota(jnp.int32, 16)
    zero16 = jnp.zeros((16,), jnp.float32)
    rows = CH // 16

    def zzr(i, _):
        zbuf[i // 8, pl.ds((i % 8) * 16, 16)] = zero16
        return 0
    lax.fori_loop(0, 128 * 8, zzr, 0)

    def one_pass(p, _):
        pltpu.sync_copy(wb.at[w, p], wbp)
        wbv = wbp[...]
        a0 = wbv[0]
        nb = wbv[1]
        a = wbv[2]
        b = wbv[3]
        base = sc * PER_SC + p * CH

        for jz in range(rows * 8 // 128):
            pltpu.sync_copy(zbuf, acc.at[pl.ds(tile * rows * 8 + jz * 128, 128)])
        plsc.subcore_barrier()

        def blk(i, __):
            e0 = pl.multiple_of(a0 + i * 16, 16)
            pltpu.sync_copy(rowi.at[pl.ds(e0, 16)], idxb)
            pltpu.sync_copy(coli.at[pl.ds(e0, 16)], colraw)
            c1 = pltpu.async_copy(phi.at[idxb], phib, sem)
            c2 = pltpu.async_copy(v.at[idxb], vb, sem)
            c3 = pltpu.async_copy(we.at[pl.ds(e0, 16)], web, sem)
            c4 = pltpu.async_copy(dirb.at[pl.ds(e0, 16)], dirbb, sem)
            colv = colraw[...]
            evec = e0 + iota
            lcol = colv - base
            ok = (evec >= a) & (evec < b) & (lcol >= 0) & (lcol < CH)
            lcolm = jnp.where(ok, lcol, DUMMY)
            for c in range(8):
                lo = lcolm[2 * c] * 8
                hi = lcolm[2 * c + 1] * 8
                colbuf[pl.ds(c * 16, 16)] = jnp.where(
                    iota < 8, lo + iota, hi + (iota - 8))
            c1.wait()
            c2.wait()
            c3.wait()
            c4.wait()

            def edge4(jj, ___):
                for dj in range(4):
                    j = jj * 4 + dj
                    r8 = j * 8
                    d0 = dirbb[j, pl.ds(0, 16)]
                    d1 = dirbb[j, pl.ds(16, 16)]
                    d2 = dirbb[j, pl.ds(32, 16)]
                    for k in range(16):
                        o = k * 16

                        def mst(off, val):
                            msgb[r8 + off // 128, pl.ds(off % 128, 16)] = val
                        mst(o, phib[j, pl.ds(o, 16)] * web[j, pl.ds(o, 16)])
                        t1 = phib[j, pl.ds(256 + o, 16)] * web[j, pl.ds(256 + o, 16)]
                        t2 = phib[j, pl.ds(512 + o, 16)] * web[j, pl.ds(512 + o, 16)]
                        vr0 = vb[j, pl.ds(o, 16)]
                        vr1 = vb[j, pl.ds(256 + o, 16)]
                        vr2 = vb[j, pl.ds(512 + o, 16)]
                        mst(256 + o, t1 * vr0 + t2 * d0)
                        mst(512 + o, t1 * vr1 + t2 * d1)
                        mst(768 + o, t1 * vr2 + t2 * d2)
                return 0
            lax.fori_loop(0, 4, edge4, 0)
            pltpu.sync_copy(msgb, acc.at[colbuf], add=True)
            return 0
        lax.fori_loop(0, nb, blk, 0)
        plsc.subcore_barrier()

        glo8 = pl.multiple_of((base + tile * rows) * 8, 8)
        pltpu.sync_copy(acc.at[pl.ds(tile * rows * 8, rows * 8)],
                        agg.at[pl.ds(glo8, rows * 8)])
        plsc.subcore_barrier()
        return 0
    lax.fori_loop(0, NCH, one_pass, 0)
    return None


_edge_sc = functools.partial(
    pl.kernel,
    mesh=plsc.VectorSubcoreMesh(core_axis_name="c", subcore_axis_name="s"),
    out_type=jax.ShapeDtypeStruct((NPAD * 8, 128), jnp.float32),
    scratch_types=[
        pltpu.VMEM((16,), jnp.int32),
        pltpu.VMEM((16,), jnp.int32),
        pltpu.VMEM((128,), jnp.int32),
        pltpu.VMEM((16, 3 * S), jnp.float32),
        pltpu.VMEM((16, 3 * S), jnp.float32),
        pltpu.VMEM((16, 3 * S), jnp.float32),
        pltpu.VMEM((16, 48), jnp.float32),
        pltpu.VMEM((128, 128), jnp.float32),
        pltpu.VMEM((128, 128), jnp.float32),
        pltpu.VMEM((16,), jnp.int32),
        pltpu.VMEM_SHARED((ACC_ROWS * 8, 128), jnp.float32),
        pltpu.SemaphoreType.DMA,
    ],
)(_edge_sc_body)


def _edge_prep(pos, row, col):
    perm = jnp.argsort(col)
    row_s = row[perm].astype(jnp.int32)
    col_s = col[perm].astype(jnp.int32)
    r_ij = pos[col_s] - pos[row_s]
    dirb, rbf_s = _rbf_pallas(r_ij)

    nodes = jnp.asarray(
        [sc * PER_SC + p * CH for sc in range(2) for p in range(NCH)],
        dtype=jnp.int32)
    cuts = jnp.searchsorted(col_s, nodes, side='left').astype(jnp.int32)
    clo = jnp.concatenate([cuts, jnp.asarray([E], dtype=jnp.int32)])

    wi = jnp.arange(NW, dtype=jnp.int32)
    sc = wi // 16
    t = wi % 16

    rows_wb = []
    for p in range(NCH):
        ca = clo[sc * NCH + p]
        cb = clo[sc * NCH + p + 1]
        a = ca + ((cb - ca) * t) // 16
        b = ca + ((cb - ca) * (t + 1)) // 16
        a0 = (a // 16) * 16
        nb = jnp.where(b > a, (b - a0 + 15) // 16, 0)
        z = jnp.zeros_like(wi)
        rows_wb.append(jnp.stack([a0, nb, a, b] + [z] * 12, axis=1))
    WB = jnp.stack(rows_wb, axis=1).astype(jnp.int32)
    return row_s, col_s, rbf_s, dirb, WB


def kernel(h, pos, edge_index, t, params):
    p = params
    row = edge_index[0]
    col = edge_index[1]

    row_s, col_s, rbf_s, dirb, WB = _edge_prep(pos, row, col)

    half = S // 2
    freqs = jnp.exp(-jnp.log(10000.0) * jnp.arange(half) / half)
    ang = t[:, None] * freqs[None, :]
    ctx = jax.nn.silu(
        jnp.concatenate([jnp.sin(ang), jnp.cos(ang)], axis=1)
        @ p['time_W1'] + p['time_b1']) @ p['time_W2'] + p['time_b2']

    s = _embed_pallas(h, p['emb_W'], p['emb_b'], ctx)
    v_cat = jnp.zeros((N, 3 * S), dtype=jnp.float32)

    for r in range(R):
        phi = _mlp2_pallas(s, p['msg_W1'][r], p['msg_b1'][r],
                           p['msg_W2'][r], p['msg_b2'][r], BN)
        we_s = _we_pallas(rbf_s, p['msg_Wr'][r], p['msg_br'][r])
        agg = _edge_sc(phi, v_cat, we_s, dirb, row_s, col_s, WB)
        agg = agg.reshape(NPAD, MC)
        s, v_cat = _update_pallas(s, v_cat, agg, ctx,
                                  p['upd_U'][r], p['upd_V'][r],
                                  p['upd_W1'][r], p['upd_b1'][r],
                                  p['upd_W2'][r], p['upd_b2'][r])

    for g in range(GEB):
        s, v_cat = _geb_pallas(s, v_cat, ctx,
                               p['geb_Wv1'][g], p['geb_Wv2'][g],
                               p['geb_W1'][g], p['geb_b1'][g],
                               p['geb_W2'][g], p['geb_b2'][g])

    equi_out, inv_out = _readout_pallas(s, v_cat, p)
    return (equi_out, inv_out)
